# baseline (speedup 1.0000x reference)
import jax
import jax.numpy as jnp
from jax.experimental import pallas as pl
from jax.experimental.pallas import tpu as pltpu

F = 8192
K = 512
TILE = 128


def _fused(mask_ref, x_ref, o_ref, prob_ref):
    @pl.when(pl.program_id(0) == 0)
    def _compute_prob():
        m = mask_ref[...]
        bits = pltpu.bitcast(m, jnp.int32)

        def vbody(_, carry):
            lo, hi = carry
            mid = (lo + hi) // 2
            c = jnp.sum((bits >= mid).astype(jnp.int32))
            big = c >= K
            return (jnp.where(big, mid, lo), jnp.where(big, hi, mid))

        t, _ = jax.lax.fori_loop(
            0, 30, vbody, (jnp.int32(0), jnp.int32(1 << 30))
        )

        gt = bits > t
        eq = bits == t
        n_gt = jnp.sum(gt.astype(jnp.int32))
        need = K - n_gt
        idx = jax.lax.broadcasted_iota(jnp.int32, (1, F), 1)

        def ibody(_, carry):
            lo, hi = carry
            mid = (lo + hi + 1) // 2
            c = jnp.sum((eq & (idx < mid)).astype(jnp.int32))
            ok = c <= need
            return (jnp.where(ok, mid, lo), jnp.where(ok, hi, mid - 1))

        cut, _ = jax.lax.fori_loop(
            0, 14, ibody, (jnp.int32(0), jnp.int32(F))
        )

        sel = gt | (eq & (idx < cut))
        maxv = jnp.max(m)
        e = jnp.where(sel, jnp.exp(m - maxv), 0.0)
        s = jnp.sum(e)
        prob_ref[...] = e * (1.0 / s)

    o_ref[...] = x_ref[...] * prob_ref[...]


def kernel(inputs, mask):
    b = inputs.shape[0]
    return pl.pallas_call(
        _fused,
        grid=(b // TILE,),
        in_specs=[
            pl.BlockSpec((1, F), lambda i: (0, 0)),
            pl.BlockSpec((TILE, F), lambda i: (i, 0)),
        ],
        out_specs=pl.BlockSpec((TILE, F), lambda i: (i, 0)),
        out_shape=jax.ShapeDtypeStruct(inputs.shape, inputs.dtype),
        scratch_shapes=[pltpu.VMEM((1, F), jnp.float32)],
    )(mask, inputs)


# speedup vs baseline: 1.0238x; 1.0238x over previous
import jax
import jax.numpy as jnp
from jax.experimental import pallas as pl
from jax.experimental.pallas import tpu as pltpu

F = 8192
K = 512
TILE = 256


def _fused(mask_ref, x_ref, o_ref, prob_ref):
    @pl.when(pl.program_id(0) == 0)
    def _compute_prob():
        m = mask_ref[...]
        bits = pltpu.bitcast(m, jnp.int32)

        def vbody(_, carry):
            lo, hi = carry
            mid = (lo + hi) // 2
            c = jnp.sum((bits >= mid).astype(jnp.int32))
            big = c >= K
            return (jnp.where(big, mid, lo), jnp.where(big, hi, mid))

        t, _ = jax.lax.fori_loop(
            0, 30, vbody, (jnp.int32(0), jnp.int32(1 << 30))
        )

        gt = bits > t
        eq = bits == t
        n_gt = jnp.sum(gt.astype(jnp.int32))
        need = K - n_gt
        idx = jax.lax.broadcasted_iota(jnp.int32, (1, F), 1)

        def ibody(_, carry):
            lo, hi = carry
            mid = (lo + hi + 1) // 2
            c = jnp.sum((eq & (idx < mid)).astype(jnp.int32))
            ok = c <= need
            return (jnp.where(ok, mid, lo), jnp.where(ok, hi, mid - 1))

        cut, _ = jax.lax.fori_loop(
            0, 14, ibody, (jnp.int32(0), jnp.int32(F))
        )

        sel = gt | (eq & (idx < cut))
        maxv = jnp.max(m)
        e = jnp.where(sel, jnp.exp(m - maxv), 0.0)
        s = jnp.sum(e)
        prob_ref[...] = e * (1.0 / s)

    o_ref[...] = x_ref[...] * prob_ref[...]


def kernel(inputs, mask):
    b = inputs.shape[0]
    return pl.pallas_call(
        _fused,
        grid=(b // TILE,),
        in_specs=[
            pl.BlockSpec((1, F), lambda i: (0, 0)),
            pl.BlockSpec((TILE, F), lambda i: (i, 0)),
        ],
        out_specs=pl.BlockSpec((TILE, F), lambda i: (i, 0)),
        out_shape=jax.ShapeDtypeStruct(inputs.shape, inputs.dtype),
        scratch_shapes=[pltpu.VMEM((1, F), jnp.float32)],
    )(mask, inputs)
